# baseline (speedup 1.0000x reference)
import functools

import jax
import jax.numpy as jnp
from jax import lax
from jax.experimental import pallas as pl
from jax.experimental.pallas import tpu as pltpu
from jax.experimental.pallas import tpu_sc as plsc

B = 16384
D = 16
NC = 2
NS = 16
NW = NC * NS
BPW = B // NW
CHUNK = 128
NCHUNK = BPW // CHUNK
GBUF = 16

_LOG1P_COEFS = (
    1.4720650109e-06,
    9.9984769750e-01,
    -4.9737321616e-01,
    3.1574731676e-01,
    -1.9035433673e-01,
    8.2691237111e-02,
    -1.7414077524e-02,
)


def _softplus(x):
    t = jnp.exp(-jnp.abs(x))
    p = jnp.full((16,), _LOG1P_COEFS[-1], jnp.float32)
    for c in reversed(_LOG1P_COEFS[:-1]):
        p = p * t + jnp.float32(c)
    return jnp.maximum(x, jnp.float32(0.0)) + p


def _theta_body(user_hbm, theta_hbm, out_hbm,
                uidx8_v, blk_v, rows_v, sem):
    wid = lax.axis_index("s") * NC + lax.axis_index("c")
    w8 = lax.shift_right_logical(wid, 3) * 8
    r8 = jnp.bitwise_and(wid, 7)
    pltpu.sync_copy(
        user_hbm.at[pl.ds(pl.multiple_of(w8, 8), 8), :], uidx8_v)

    iota16 = lax.iota(jnp.int32, 16)
    r8vec = jnp.full((16,), r8, jnp.int32)
    NG = BPW // GBUF

    def fire(g, slot):
        u16 = plsc.load_gather(uidx8_v, [r8vec, g * GBUF + iota16])
        grp16 = lax.shift_right_logical(u16, 7)
        for t in range(GBUF):
            start = grp16[t] * 128
            pltpu.async_copy(
                theta_hbm.at[:, pl.ds(pl.multiple_of(start, 128), 128)],
                blk_v.at[slot, t], sem)
        return u16

    fire(0, 0)
    fire(1, 1)

    def step(g, carry):
        s = lax.rem(g, 3)
        u16 = plsc.load_gather(uidx8_v, [r8vec, g * GBUF + iota16])
        lane16 = jnp.bitwise_and(u16, 127)

        @pl.when(g + 2 < NG)
        def _fire_next():
            fire(g + 2, lax.rem(g + 2, 3))

        for t in range(GBUF):
            pltpu.make_async_copy(theta_hbm.at[:, pl.ds(0, 128)],
                                  blk_v.at[0, t], sem).wait()
        svec = jnp.full((16,), s, jnp.int32)
        for t in range(GBUF):
            tvec = jnp.full((16,), t, jnp.int32)
            lvec = jnp.full((16,), lane16[t], jnp.int32)
            col = plsc.load_gather(blk_v, [svec, tvec, iota16, lvec])
            rows_v[pl.ds((g * GBUF + t) * D, D)] = col
        return carry

    lax.fori_loop(0, NG, step, 0)

    pltpu.sync_copy(rows_v, out_hbm.at[pl.ds(wid * (BPW * D), BPW * D)])


_theta_gather = functools.partial(
    pl.kernel,
    out_type=jax.ShapeDtypeStruct((B * D,), jnp.float32),
    mesh=plsc.VectorSubcoreMesh(core_axis_name="c", subcore_axis_name="s"),
    compiler_params=pltpu.CompilerParams(
        needs_layout_passes=False, use_tc_tiling_on_sc=True,
        skip_device_barrier=True),
    scratch_types=[
        pltpu.VMEM((8, BPW), jnp.int32),
        pltpu.VMEM((3, GBUF, D, 128), jnp.float32),
        pltpu.VMEM((BPW * D,), jnp.float32),
        pltpu.SemaphoreType.DMA,
    ],
)(_theta_body)


def _mirt_body(item_hbm, thg_hbm, a_hbm, b_hbm, out_hbm,
               iidx_v, th_v, a_v, b_v, out_v, sem_t, sem_a, sem_b):
    wid = lax.axis_index("s") * NC + lax.axis_index("c")

    pltpu.sync_copy(item_hbm.at[wid], iidx_v)
    cp_th = pltpu.async_copy(
        thg_hbm.at[pl.ds(wid * (BPW * D), BPW * D)], th_v, sem_t)

    copies = []
    for k in range(NCHUNK):
        copies.append(pltpu.async_copy(a_hbm.at[iidx_v.at[k]],
                                       a_v.at[k], sem_a))
        copies.append(pltpu.async_copy(b_hbm.at[iidx_v.at[k]],
                                       b_v.at[k], sem_b))
    cp_th.wait()
    for cp in copies:
        cp.wait()

    iota16 = lax.iota(jnp.int32, 16)

    for k in range(NCHUNK):
        kvec = jnp.full((16,), k, jnp.int32)

        def block(j, carry, k=k, kvec=kvec):
            rows = j * 16 + iota16
            flat0 = (k * CHUNK + rows) * D
            acc = -plsc.load_gather(b_v, [kvec, rows])
            for d in range(D):
                dvec = jnp.full((16,), d, jnp.int32)
                th = plsc.load_gather(th_v, [flat0 + d])
                av = plsc.load_gather(a_v, [kvec, rows, dvec])
                acc = acc + _softplus(av) * th
            res = jnp.float32(1.0) / (jnp.float32(1.0) + jnp.exp(-acc))
            out_v[pl.ds(k * CHUNK + j * 16, 16)] = res
            return carry

        lax.fori_loop(0, CHUNK // 16, block, 0)

    pltpu.sync_copy(out_v, out_hbm.at[wid])


_mirt = functools.partial(
    pl.kernel,
    out_type=jax.ShapeDtypeStruct((NW, BPW), jnp.float32),
    mesh=plsc.VectorSubcoreMesh(core_axis_name="c", subcore_axis_name="s"),
    compiler_params=pltpu.CompilerParams(
        needs_layout_passes=False, use_tc_tiling_on_sc=False,
        skip_device_barrier=True),
    scratch_types=[
        pltpu.VMEM((NCHUNK, CHUNK), jnp.int32),
        pltpu.VMEM((BPW * D,), jnp.float32),
        pltpu.VMEM((NCHUNK, CHUNK, D), jnp.float32),
        pltpu.VMEM((NCHUNK, CHUNK), jnp.float32),
        pltpu.VMEM((BPW,), jnp.float32),
        pltpu.SemaphoreType.DMA,
        pltpu.SemaphoreType.DMA,
        pltpu.SemaphoreType.DMA,
    ],
)(_mirt_body)


def kernel(user, item, theta_table, a_table, b_table):
    user = user.astype(jnp.int32).reshape(NW, BPW)
    item = item.astype(jnp.int32).reshape(NW, NCHUNK, CHUNK)
    theta_t = theta_table.T
    b_flat = b_table.reshape(-1)
    thg = _theta_gather(user, theta_t)
    out = _mirt(item, thg, a_table, b_flat)
    return out.reshape(B)


# speedup vs baseline: 1.0132x; 1.0132x over previous
import functools

import jax
import jax.numpy as jnp
from jax import lax
from jax.experimental import pallas as pl
from jax.experimental.pallas import tpu as pltpu
from jax.experimental.pallas import tpu_sc as plsc

B = 16384
D = 16
NC = 2
NS = 16
NW = NC * NS
BPW = B // NW
CHUNK = 128
NCHUNK = BPW // CHUNK
GBUF = 16

_LOG1P_COEFS = (
    1.4720650109e-06,
    9.9984769750e-01,
    -4.9737321616e-01,
    3.1574731676e-01,
    -1.9035433673e-01,
    8.2691237111e-02,
    -1.7414077524e-02,
)


def _softplus(x):
    t = jnp.exp(-jnp.abs(x))
    p = jnp.full((16,), _LOG1P_COEFS[-1], jnp.float32)
    for c in reversed(_LOG1P_COEFS[:-1]):
        p = p * t + jnp.float32(c)
    return jnp.maximum(x, jnp.float32(0.0)) + p


def _theta_body(user_hbm, theta_hbm, out_hbm,
                uidx8_v, blk_v, rows_v, sem):
    wid = lax.axis_index("s") * NC + lax.axis_index("c")
    w8 = lax.shift_right_logical(wid, 3) * 8
    r8 = jnp.bitwise_and(wid, 7)
    pltpu.sync_copy(
        user_hbm.at[pl.ds(pl.multiple_of(w8, 8), 8), :], uidx8_v)

    iota16 = lax.iota(jnp.int32, 16)
    r8vec = jnp.full((16,), r8, jnp.int32)
    NG = BPW // GBUF

    def fire(g, slot):
        u16 = plsc.load_gather(uidx8_v, [r8vec, g * GBUF + iota16])
        grp16 = lax.shift_right_logical(u16, 7)
        for t in range(GBUF):
            start = grp16[t] * 128
            pltpu.async_copy(
                theta_hbm.at[:, pl.ds(pl.multiple_of(start, 128), 128)],
                blk_v.at[slot, t], sem)
        return u16

    fire(0, 0)

    def step(g, carry):
        s = jnp.bitwise_and(g, 1)
        u16 = plsc.load_gather(uidx8_v, [r8vec, g * GBUF + iota16])
        lane16 = jnp.bitwise_and(u16, 127)

        @pl.when(g + 1 < NG)
        def _fire_next():
            fire(g + 1, 1 - s)

        for t in range(GBUF):
            pltpu.make_async_copy(theta_hbm.at[:, pl.ds(0, 128)],
                                  blk_v.at[0, t], sem).wait()
        svec = jnp.full((16,), s, jnp.int32)
        for t in range(GBUF):
            tvec = jnp.full((16,), t, jnp.int32)
            lvec = jnp.full((16,), lane16[t], jnp.int32)
            col = plsc.load_gather(blk_v, [svec, tvec, iota16, lvec])
            rows_v[pl.ds((g * GBUF + t) * D, D)] = col
        return carry

    lax.fori_loop(0, NG, step, 0)

    pltpu.sync_copy(rows_v, out_hbm.at[pl.ds(wid * (BPW * D), BPW * D)])


_theta_gather = functools.partial(
    pl.kernel,
    out_type=jax.ShapeDtypeStruct((B * D,), jnp.float32),
    mesh=plsc.VectorSubcoreMesh(core_axis_name="c", subcore_axis_name="s"),
    compiler_params=pltpu.CompilerParams(
        needs_layout_passes=False, use_tc_tiling_on_sc=True,
        skip_device_barrier=True),
    scratch_types=[
        pltpu.VMEM((8, BPW), jnp.int32),
        pltpu.VMEM((2, GBUF, D, 128), jnp.float32),
        pltpu.VMEM((BPW * D,), jnp.float32),
        pltpu.SemaphoreType.DMA,
    ],
)(_theta_body)


def _mirt_body(item_hbm, thg_hbm, a_hbm, b_hbm, out_hbm,
               iidx_v, th_v, a_v, b_v, out_v, sem_t, sem_a, sem_b):
    wid = lax.axis_index("s") * NC + lax.axis_index("c")

    pltpu.sync_copy(item_hbm.at[wid], iidx_v)
    cp_th = pltpu.async_copy(
        thg_hbm.at[pl.ds(wid * (BPW * D), BPW * D)], th_v, sem_t)

    copies = []
    for k in range(NCHUNK):
        copies.append(pltpu.async_copy(a_hbm.at[iidx_v.at[k]],
                                       a_v.at[k], sem_a))
        copies.append(pltpu.async_copy(b_hbm.at[iidx_v.at[k]],
                                       b_v.at[k], sem_b))
    cp_th.wait()
    for cp in copies:
        cp.wait()

    iota16 = lax.iota(jnp.int32, 16)

    for k in range(NCHUNK):
        kvec = jnp.full((16,), k, jnp.int32)

        def block(j, carry, k=k, kvec=kvec):
            rows = j * 16 + iota16
            flat0 = (k * CHUNK + rows) * D
            acc = -plsc.load_gather(b_v, [kvec, rows])
            for d in range(D):
                dvec = jnp.full((16,), d, jnp.int32)
                th = plsc.load_gather(th_v, [flat0 + d])
                av = plsc.load_gather(a_v, [kvec, rows, dvec])
                acc = acc + _softplus(av) * th
            res = jnp.float32(1.0) / (jnp.float32(1.0) + jnp.exp(-acc))
            out_v[pl.ds(k * CHUNK + j * 16, 16)] = res
            return carry

        lax.fori_loop(0, CHUNK // 16, block, 0)

    pltpu.sync_copy(out_v, out_hbm.at[wid])


_mirt = functools.partial(
    pl.kernel,
    out_type=jax.ShapeDtypeStruct((NW, BPW), jnp.float32),
    mesh=plsc.VectorSubcoreMesh(core_axis_name="c", subcore_axis_name="s"),
    compiler_params=pltpu.CompilerParams(
        needs_layout_passes=False, use_tc_tiling_on_sc=False,
        skip_device_barrier=True),
    scratch_types=[
        pltpu.VMEM((NCHUNK, CHUNK), jnp.int32),
        pltpu.VMEM((BPW * D,), jnp.float32),
        pltpu.VMEM((NCHUNK, CHUNK, D), jnp.float32),
        pltpu.VMEM((NCHUNK, CHUNK), jnp.float32),
        pltpu.VMEM((BPW,), jnp.float32),
        pltpu.SemaphoreType.DMA,
        pltpu.SemaphoreType.DMA,
        pltpu.SemaphoreType.DMA,
    ],
)(_mirt_body)


def kernel(user, item, theta_table, a_table, b_table):
    user = user.astype(jnp.int32).reshape(NW, BPW)
    item = item.astype(jnp.int32).reshape(NW, NCHUNK, CHUNK)
    theta_t = theta_table.T
    b_flat = b_table.reshape(-1)
    thg = _theta_gather(user, theta_t)
    out = _mirt(item, thg, a_table, b_flat)
    return out.reshape(B)
